# baseline (speedup 1.0000x reference)
import functools

import jax
import jax.numpy as jnp
from jax import lax
from jax.experimental import pallas as pl
from jax.experimental.pallas import tpu as pltpu
from jax.experimental.pallas import tpu_sc as plsc

N_E = 1024
E_DIM = 256
BATCH = 16
S = 1024

_PREC = lax.Precision.DEFAULT

NC = 2
NS = 16
NW = NC * NS
N_POS = BATCH * S
B_PER_W = N_POS // NW
CHUNK = 128
N_CHUNKS = B_PER_W // CHUNK


def _dist_body(z_ref, e_ref, idx_ref, loss_ref):
    b = pl.program_id(0)
    z_b = z_ref[0]
    e = e_ref[...]

    en = jnp.sum(e * e, axis=1, keepdims=True)
    prod = lax.dot_general(
        e, z_b, (((1,), (0,)), ((), ())),
        preferred_element_type=jnp.float32, precision=_PREC)
    d = en - 2.0 * prod

    minv = jnp.min(d, axis=0)
    idx = jnp.argmin(d, axis=0).astype(jnp.int32)
    idx_ref[0, 0] = idx

    zn = jnp.sum(z_b * z_b, axis=0)
    part = jnp.sum(minv + zn)

    @pl.when(b == 0)
    def _():
        loss_ref[0, 0] = 0.0

    loss_ref[0, 0] += part


_sc_mesh = plsc.VectorSubcoreMesh(core_axis_name="c", subcore_axis_name="s")


@functools.partial(
    pl.kernel,
    out_type=jax.ShapeDtypeStruct((N_POS, E_DIM), jnp.float32),
    mesh=_sc_mesh,
    scratch_types=[
        pltpu.VMEM((N_CHUNKS, CHUNK), jnp.int32),
        pltpu.VMEM((CHUNK, E_DIM), jnp.float32),
        pltpu.VMEM((CHUNK, E_DIM), jnp.float32),
        pltpu.SemaphoreType.DMA,
        pltpu.SemaphoreType.DMA,
    ],
)
def _sc_gather(table_hbm, idx_hbm, out_hbm, idx_v, rows0, rows1, sem0, sem1):
    wid = lax.axis_index("s") * NC + lax.axis_index("c")
    pltpu.sync_copy(idx_hbm.at[pl.ds(wid * N_CHUNKS, N_CHUNKS)], idx_v)
    bufs = (rows0, rows1)
    sems = (sem0, sem1)
    copies = [pltpu.async_copy(table_hbm.at[idx_v.at[0]], bufs[0], sems[0])]
    for j in range(N_CHUNKS):
        if j + 1 < N_CHUNKS:
            copies.append(pltpu.async_copy(
                table_hbm.at[idx_v.at[j + 1]], bufs[(j + 1) % 2],
                sems[(j + 1) % 2]))
        copies[j].wait()
        pltpu.sync_copy(
            bufs[j % 2],
            out_hbm.at[pl.ds(wid * B_PER_W + j * CHUNK, CHUNK)])


@jax.jit
def kernel(z, embedding):
    z3 = z.reshape(BATCH, E_DIM, S)
    idx3, loss_acc = pl.pallas_call(
        _dist_body,
        grid=(BATCH,),
        in_specs=[
            pl.BlockSpec((1, E_DIM, S), lambda b: (b, 0, 0)),
            pl.BlockSpec((N_E, E_DIM), lambda b: (0, 0)),
        ],
        out_specs=[
            pl.BlockSpec((1, 1, S), lambda b: (b, 0, 0)),
            pl.BlockSpec(memory_space=pltpu.SMEM, block_shape=(1, 1),
                         index_map=lambda b: (0, 0)),
        ],
        out_shape=[
            jax.ShapeDtypeStruct((BATCH, 1, S), jnp.int32),
            jax.ShapeDtypeStruct((1, 1), jnp.float32),
        ],
    )(z3, embedding)

    idx2 = idx3.reshape(N_POS // CHUNK, CHUNK)
    zq_sc = _sc_gather(embedding, idx2)
    z_q_out = zq_sc.reshape(BATCH, 32, 32, E_DIM).transpose(0, 3, 1, 2)
    idx_out = idx3.reshape(BATCH, 32, 32)
    loss = loss_acc[0, 0] / jnp.float32(BATCH * S * E_DIM / 2)
    return (z_q_out, loss, idx_out)


# speedup vs baseline: 1.0119x; 1.0119x over previous
import functools

import jax
import jax.numpy as jnp
from jax import lax
from jax.experimental import pallas as pl
from jax.experimental.pallas import tpu as pltpu
from jax.experimental.pallas import tpu_sc as plsc

N_E = 1024
E_DIM = 256
BATCH = 16
S = 1024

_PREC = lax.Precision.DEFAULT

NC = 2
NS = 16
NW = NC * NS
N_POS = BATCH * S
B_PER_W = N_POS // NW
CHUNK = 128
N_CHUNKS = B_PER_W // CHUNK


def _dist_body(z_ref, e_ref, idx_ref, loss_ref):
    b = pl.program_id(0)
    z_b = z_ref[0]
    e = e_ref[...]

    en = jnp.sum(e * e, axis=1, keepdims=True)
    prod = lax.dot_general(
        e, z_b, (((1,), (0,)), ((), ())),
        preferred_element_type=jnp.float32, precision=_PREC)
    d = en - 2.0 * prod

    minv = jnp.min(d, axis=0)
    idx = jnp.argmin(d, axis=0).astype(jnp.int32)
    idx_ref[0, 0] = idx

    zn = jnp.sum(z_b * z_b, axis=0)
    part = jnp.sum(minv + zn)

    @pl.when(b == 0)
    def _():
        loss_ref[0, 0] = 0.0

    loss_ref[0, 0] += part


_sc_mesh = plsc.VectorSubcoreMesh(core_axis_name="c", subcore_axis_name="s")


_N_BUF = 3


@functools.partial(
    pl.kernel,
    out_type=jax.ShapeDtypeStruct((N_POS, E_DIM), jnp.float32),
    mesh=_sc_mesh,
    scratch_types=[
        pltpu.VMEM((N_CHUNKS, CHUNK), jnp.int32),
        pltpu.VMEM((CHUNK, E_DIM), jnp.float32),
        pltpu.VMEM((CHUNK, E_DIM), jnp.float32),
        pltpu.VMEM((CHUNK, E_DIM), jnp.float32),
        pltpu.SemaphoreType.DMA,
        pltpu.SemaphoreType.DMA,
        pltpu.SemaphoreType.DMA,
        pltpu.SemaphoreType.DMA,
        pltpu.SemaphoreType.DMA,
        pltpu.SemaphoreType.DMA,
    ],
)
def _sc_gather(table_hbm, idx_hbm, out_hbm, idx_v, rows0, rows1, rows2,
               gs0, gs1, gs2, ws0, ws1, ws2):
    wid = lax.axis_index("s") * NC + lax.axis_index("c")
    pltpu.sync_copy(idx_hbm.at[pl.ds(wid * N_CHUNKS, N_CHUNKS)], idx_v)
    bufs = (rows0, rows1, rows2)
    gsems = (gs0, gs1, gs2)
    wsems = (ws0, ws1, ws2)
    gathers = [None] * N_CHUNKS
    writes = [None] * N_CHUNKS
    for j in range(min(_N_BUF, N_CHUNKS)):
        gathers[j] = pltpu.async_copy(
            table_hbm.at[idx_v.at[j]], bufs[j % _N_BUF], gsems[j % _N_BUF])
    for j in range(N_CHUNKS):
        gathers[j].wait()
        writes[j] = pltpu.async_copy(
            bufs[j % _N_BUF],
            out_hbm.at[pl.ds(wid * B_PER_W + j * CHUNK, CHUNK)],
            wsems[j % _N_BUF])
        nxt = j + _N_BUF
        if nxt < N_CHUNKS:
            writes[j].wait()
            gathers[nxt] = pltpu.async_copy(
                table_hbm.at[idx_v.at[nxt]], bufs[nxt % _N_BUF],
                gsems[nxt % _N_BUF])
    for j in range(max(0, N_CHUNKS - _N_BUF), N_CHUNKS):
        writes[j].wait()


@jax.jit
def kernel(z, embedding):
    z3 = z.reshape(BATCH, E_DIM, S)
    idx3, loss_acc = pl.pallas_call(
        _dist_body,
        grid=(BATCH,),
        in_specs=[
            pl.BlockSpec((1, E_DIM, S), lambda b: (b, 0, 0)),
            pl.BlockSpec((N_E, E_DIM), lambda b: (0, 0)),
        ],
        out_specs=[
            pl.BlockSpec((1, 1, S), lambda b: (b, 0, 0)),
            pl.BlockSpec(memory_space=pltpu.SMEM, block_shape=(1, 1),
                         index_map=lambda b: (0, 0)),
        ],
        out_shape=[
            jax.ShapeDtypeStruct((BATCH, 1, S), jnp.int32),
            jax.ShapeDtypeStruct((1, 1), jnp.float32),
        ],
    )(z3, embedding)

    idx2 = idx3.reshape(N_POS // CHUNK, CHUNK)
    zq_sc = _sc_gather(embedding, idx2)
    z_q_out = zq_sc.reshape(BATCH, 32, 32, E_DIM).transpose(0, 3, 1, 2)
    idx_out = idx3.reshape(BATCH, 32, 32)
    loss = loss_acc[0, 0] / jnp.float32(BATCH * S * E_DIM / 2)
    return (z_q_out, loss, idx_out)
